# Initial kernel scaffold; baseline (speedup 1.0000x reference)
import jax
import jax.numpy as jnp
from jax.experimental import pallas as pl


def kernel(coord, feat, offset, W1, W2, W3, W4, W5, U5_l1_W, U5_l1_b, U5_l2_W, U5_l2_b, U4_l1_W, U4_l1_b, U4_l2_W, U4_l2_b, U3_l1_W, U3_l1_b, U3_l2_W, U3_l2_b, U2_l1_W, U2_l1_b, U2_l2_W, U2_l2_b, U1_l1_W, U1_l1_b, U1_l2_W, U1_l2_b, C1_W, C1_b, C2_W, C2_b):
    raise NotImplementedError("write your pallas kernel here")


# speedup vs baseline: 4.3083x; 4.3083x over previous
import functools

import jax
import jax.numpy as jnp
from jax import lax
from jax.experimental import pallas as pl
from jax.experimental.pallas import tpu as pltpu
from jax.experimental.pallas import tpu_sc as plsc

F32 = jnp.float32


def _rup(x, m):
    return (x + m - 1) // m * m


def _lin_bn_relu_body(x_ref, w_ref, b_ref, o_ref):
    h = jnp.dot(x_ref[...], w_ref[...], preferred_element_type=F32) + b_ref[...]
    mu = jnp.mean(h, axis=0, keepdims=True)
    var = jnp.mean(h * h, axis=0, keepdims=True) - mu * mu
    o_ref[...] = jnp.maximum((h - mu) * lax.rsqrt(var + 1e-5), 0.0)


def _lin_bn_relu(x, w, b):
    n, cout = x.shape[0], w.shape[1]
    return pl.pallas_call(
        _lin_bn_relu_body,
        out_shape=jax.ShapeDtypeStruct((n, cout), F32),
    )(x, w, b.reshape(1, cout))


def _zx_body(x_ref, wx_ref, z_ref):
    z_ref[...] = jnp.dot(x_ref[...], wx_ref[...], preferred_element_type=F32)


def _zx(x, wx):
    n, cout = x.shape[0], wx.shape[1]
    return pl.pallas_call(
        _zx_body,
        out_shape=jax.ShapeDtypeStruct((n, cout), F32),
    )(x, wx)


def _knn_body(q_ref, rt_ref, idx_ref, d2_ref, *, k, nsrc):
    q = q_ref[...]
    rt = rt_ref[...]
    d2 = (jnp.sum(q * q, axis=1, keepdims=True)
          - 2.0 * jnp.dot(q.astype(jnp.bfloat16), rt.astype(jnp.bfloat16),
                          preferred_element_type=F32)
          + jnp.sum(rt * rt, axis=0, keepdims=True))
    iota = lax.broadcasted_iota(jnp.int32, d2.shape, 1)
    idx_cols, d_cols = [], []
    for _ in range(k):
        m = jnp.min(d2, axis=1, keepdims=True)
        j = jnp.min(jnp.where(d2 == m, iota, nsrc), axis=1, keepdims=True)
        idx_cols.append(j)
        d_cols.append(m)
        d2 = jnp.where(iota == j, jnp.inf, d2)
    idx_ref[...] = jnp.concatenate(idx_cols, axis=1)
    d2_ref[...] = jnp.concatenate(d_cols, axis=1)


def _knn(q, r, k):
    m, nsrc = q.shape[0], r.shape[0]
    cap = max(8, (int(1.5e6) // max(nsrc, 1)) // 8 * 8)
    grid = -(-m // cap)
    bq = _rup(-(-m // grid), 8)
    mp = bq * grid
    if mp > m:
        q = jnp.concatenate(
            [q, jnp.full((mp - m, q.shape[1]), 1e9, F32)], axis=0)
    rt = r.T
    idx, d2 = pl.pallas_call(
        functools.partial(_knn_body, k=k, nsrc=nsrc),
        grid=(grid,),
        in_specs=[pl.BlockSpec((bq, 3), lambda i: (i, 0)),
                  pl.BlockSpec((3, nsrc), lambda i: (0, 0))],
        out_specs=[pl.BlockSpec((bq, k), lambda i: (i, 0)),
                   pl.BlockSpec((bq, k), lambda i: (i, 0))],
        out_shape=[jax.ShapeDtypeStruct((mp, k), jnp.int32),
                   jax.ShapeDtypeStruct((mp, k), F32)],
    )(q, rt)
    return idx[:m], d2[:m]


@functools.lru_cache(maxsize=None)
def _make_sc_gather(v, d, b):
    info = plsc.get_sparse_core_info()
    nw = info.num_cores * info.num_subcores
    bw = b // nw
    mesh = plsc.VectorSubcoreMesh(core_axis_name="c", subcore_axis_name="s")

    @functools.partial(
        pl.kernel, mesh=mesh,
        out_type=jax.ShapeDtypeStruct((b, d), F32),
        scratch_types=[pltpu.VMEM((bw,), jnp.int32),
                       pltpu.VMEM((128, d), F32),
                       pltpu.SemaphoreType.DMA],
    )
    def gather(table_hbm, idx_hbm, out_hbm, idx_v, rows_v, sem):
        wid = lax.axis_index("s") * info.num_cores + lax.axis_index("c")
        base = wid * bw
        pltpu.sync_copy(idx_hbm.at[pl.ds(base, bw)], idx_v)
        for j0 in range(0, bw, 128):
            sz = min(128, bw - j0)
            pltpu.async_copy(table_hbm.at[idx_v.at[pl.ds(j0, sz)]],
                             rows_v.at[pl.ds(0, sz)], sem).wait()
            pltpu.sync_copy(rows_v.at[pl.ds(0, sz)],
                            out_hbm.at[pl.ds(base + j0, sz)])

    return gather


def _gather_rows(table, flat_idx):
    n = flat_idx.shape[0]
    b = _rup(n, 256)
    if b > n:
        flat_idx = jnp.concatenate(
            [flat_idx, jnp.zeros((b - n,), jnp.int32)], axis=0)
    d = table.shape[1]
    dp = _rup(d, 128)
    if dp > d:
        table = jnp.pad(table, ((0, 0), (0, dp - d)))
    out = _make_sc_gather(table.shape[0], dp, b)(table, flat_idx)
    return out[:n, :d]


def _td_stats_body(g_ref, q_ref, wp_ref, hmax_ref, s1_ref, s2_ref, *, bm, k, cout):
    i = pl.program_id(0)
    g = g_ref[...]
    dp = g[:, :, :3] - q_ref[...][:, None, :]
    hp = jnp.dot(dp.reshape(bm * k, 3).astype(jnp.bfloat16),
                 wp_ref[...].astype(jnp.bfloat16),
                 preferred_element_type=F32)
    h = hp.reshape(bm, k, cout) + g[:, :, 3:]
    hmax_ref[...] = jnp.max(h, axis=1)
    ps1 = jnp.sum(jnp.sum(h, axis=1), axis=0, keepdims=True)
    ps2 = jnp.sum(jnp.sum(h * h, axis=1), axis=0, keepdims=True)

    @pl.when(i == 0)
    def _init():
        s1_ref[...] = ps1
        s2_ref[...] = ps2

    @pl.when(i > 0)
    def _acc():
        s1_ref[...] += ps1
        s2_ref[...] += ps2


def _td_norm_body(hmax_ref, s1_ref, s2_ref, o_ref, *, n):
    mu = s1_ref[...] / n
    var = s2_ref[...] / n - mu * mu
    o_ref[...] = jnp.maximum((hmax_ref[...] - mu) * lax.rsqrt(var + 1e-5), 0.0)


def _td_fuse(g, q, wp):
    m, k, dg = g.shape
    cout = dg - 3
    nb = max(1, -(-(m * k * dg * 4) // (3 * 2**20)))
    mp = _rup(m, 8 * nb)
    if mp > m:
        g = jnp.pad(g, ((0, mp - m), (0, 0), (0, 0)))
        q = jnp.pad(q, ((0, mp - m), (0, 0)))
    bm = mp // nb
    hmax, s1, s2 = pl.pallas_call(
        functools.partial(_td_stats_body, bm=bm, k=k, cout=cout),
        grid=(nb,),
        in_specs=[pl.BlockSpec((bm, k, dg), lambda i: (i, 0, 0)),
                  pl.BlockSpec((bm, 3), lambda i: (i, 0)),
                  pl.BlockSpec((3, cout), lambda i: (0, 0))],
        out_specs=[pl.BlockSpec((bm, cout), lambda i: (i, 0)),
                   pl.BlockSpec((1, cout), lambda i: (0, 0)),
                   pl.BlockSpec((1, cout), lambda i: (0, 0))],
        out_shape=[jax.ShapeDtypeStruct((mp, cout), F32),
                   jax.ShapeDtypeStruct((1, cout), F32),
                   jax.ShapeDtypeStruct((1, cout), F32)],
    )(g, q, wp)
    return pl.pallas_call(
        functools.partial(_td_norm_body, n=m * k),
        out_shape=jax.ShapeDtypeStruct((mp, cout), F32),
    )(hmax, s1, s2)[:m]


def _tu_fuse_body(x_ref, w_ref, b_ref, g0_ref, g1_ref, g2_ref, d2_ref, o_ref):
    h = jnp.dot(x_ref[...], w_ref[...], preferred_element_type=F32) + b_ref[...]
    mu = jnp.mean(h, axis=0, keepdims=True)
    var = jnp.mean(h * h, axis=0, keepdims=True) - mu * mu
    a = jnp.maximum((h - mu) * lax.rsqrt(var + 1e-5), 0.0)
    d2 = jnp.maximum(d2_ref[...], 0.0)
    w = 1.0 / (jnp.sqrt(d2) + 1e-8)
    w = w / jnp.sum(w, axis=1, keepdims=True)
    interp = (g0_ref[...] * w[:, 0:1] + g1_ref[...] * w[:, 1:2]
              + g2_ref[...] * w[:, 2:3])
    o_ref[...] = a + interp


def _tu_fuse(x, w, b, g, d2):
    n, cout = x.shape[0], w.shape[1]
    return pl.pallas_call(
        _tu_fuse_body,
        out_shape=jax.ShapeDtypeStruct((n, cout), F32),
    )(x, w, b.reshape(1, cout), g[:, 0, :], g[:, 1, :], g[:, 2, :], d2)


def _u5_body(x_ref, a_ref, bmat_ref, b1_ref, l2w_ref, l2b_ref, o_ref):
    x = x_ref[...]
    gmean = jnp.mean(x, axis=0, keepdims=True)
    g = jnp.maximum(
        jnp.dot(gmean, l2w_ref[...], preferred_element_type=F32) + l2b_ref[...],
        0.0)
    h = (jnp.dot(x, a_ref[...], preferred_element_type=F32)
         + jnp.dot(g, bmat_ref[...], preferred_element_type=F32)
         + b1_ref[...])
    mu = jnp.mean(h, axis=0, keepdims=True)
    var = jnp.mean(h * h, axis=0, keepdims=True) - mu * mu
    o_ref[...] = jnp.maximum((h - mu) * lax.rsqrt(var + 1e-5), 0.0)


def _u5(x5, l1w, l1b, l2w, l2b):
    cnt, c = x5.shape
    cout = l1w.shape[1]
    return pl.pallas_call(
        _u5_body,
        out_shape=jax.ShapeDtypeStruct((cnt, cout), F32),
    )(x5, l1w[:c], l1w[c:], l1b.reshape(1, cout), l2w, l2b.reshape(1, -1))


def _head_body(x_ref, w1_ref, b1_ref, w2_ref, b2_ref, o_ref):
    h = jnp.dot(x_ref[...], w1_ref[...], preferred_element_type=F32) + b1_ref[...]
    mu = jnp.mean(h, axis=0, keepdims=True)
    var = jnp.mean(h * h, axis=0, keepdims=True) - mu * mu
    h = jnp.maximum((h - mu) * lax.rsqrt(var + 1e-5), 0.0)
    o_ref[...] = jnp.dot(h, w2_ref[...], preferred_element_type=F32) + b2_ref[...]


def _head(x, w1, b1, w2, b2):
    n, cout = x.shape[0], w2.shape[1]
    return pl.pallas_call(
        _head_body,
        out_shape=jax.ShapeDtypeStruct((n, cout), F32),
    )(x, w1, b1.reshape(1, -1), w2, b2.reshape(1, cout))


def _transition_down(p, x, w, stride, k):
    nsrc = p.shape[0]
    m = nsrc // stride
    cout = w.shape[1]
    q = p[::stride][:m]
    zx = _zx(x, w[3:])
    table = jnp.concatenate([p, zx], axis=1)
    idx, _ = _knn(q, p, k)
    g = _gather_rows(table, idx.reshape(-1)).reshape(m, k, 3 + cout)
    return q, _td_fuse(g, q, w[:3])


def _transition_up(p1, x1, p2, x2, l1w, l1b, l2w, l2b):
    b2 = _lin_bn_relu(x2, l2w, l2b)
    idx, d2 = _knn(p1, p2, 3)
    g = _gather_rows(b2, idx.reshape(-1)).reshape(p1.shape[0], 3, -1)
    return _tu_fuse(x1, l1w, l1b, g, d2)


def kernel(coord, feat, offset, W1, W2, W3, W4, W5,
           U5_l1_W, U5_l1_b, U5_l2_W, U5_l2_b,
           U4_l1_W, U4_l1_b, U4_l2_W, U4_l2_b,
           U3_l1_W, U3_l1_b, U3_l2_W, U3_l2_b,
           U2_l1_W, U2_l1_b, U2_l2_W, U2_l2_b,
           U1_l1_W, U1_l1_b, U1_l2_W, U1_l2_b,
           C1_W, C1_b, C2_W, C2_b):
    del offset
    p1 = coord
    x1 = _lin_bn_relu(feat, W1, jnp.zeros((W1.shape[1],), F32))
    p2, x2 = _transition_down(p1, x1, W2, 4, 16)
    p3, x3 = _transition_down(p2, x2, W3, 4, 16)
    p4, x4 = _transition_down(p3, x3, W4, 4, 16)
    p5, x5 = _transition_down(p4, x4, W5, 4, 16)
    x5 = _u5(x5, U5_l1_W, U5_l1_b, U5_l2_W, U5_l2_b)
    x4 = _transition_up(p4, x4, p5, x5, U4_l1_W, U4_l1_b, U4_l2_W, U4_l2_b)
    x3 = _transition_up(p3, x3, p4, x4, U3_l1_W, U3_l1_b, U3_l2_W, U3_l2_b)
    x2 = _transition_up(p2, x2, p3, x3, U2_l1_W, U2_l1_b, U2_l2_W, U2_l2_b)
    x1 = _transition_up(p1, x1, p2, x2, U1_l1_W, U1_l1_b, U1_l2_W, U1_l2_b)
    return _head(x1, C1_W, C1_b, C2_W, C2_b)


# speedup vs baseline: 4.3849x; 1.0178x over previous
import functools

import jax
import jax.numpy as jnp
from jax import lax
from jax.experimental import pallas as pl
from jax.experimental.pallas import tpu as pltpu
from jax.experimental.pallas import tpu_sc as plsc

F32 = jnp.float32


def _rup(x, m):
    return (x + m - 1) // m * m


def _lin_bn_relu_body(x_ref, w_ref, b_ref, o_ref):
    h = jnp.dot(x_ref[...], w_ref[...], preferred_element_type=F32) + b_ref[...]
    mu = jnp.mean(h, axis=0, keepdims=True)
    var = jnp.mean(h * h, axis=0, keepdims=True) - mu * mu
    o_ref[...] = jnp.maximum((h - mu) * lax.rsqrt(var + 1e-5), 0.0)


def _lin_bn_relu(x, w, b):
    n, cout = x.shape[0], w.shape[1]
    return pl.pallas_call(
        _lin_bn_relu_body,
        out_shape=jax.ShapeDtypeStruct((n, cout), F32),
    )(x, w, b.reshape(1, cout))


def _zx_body(x_ref, wx_ref, z_ref):
    z_ref[...] = jnp.dot(x_ref[...], wx_ref[...], preferred_element_type=F32)


def _zx(x, wx):
    n, cout = x.shape[0], wx.shape[1]
    return pl.pallas_call(
        _zx_body,
        out_shape=jax.ShapeDtypeStruct((n, cout), F32),
    )(x, wx)


def _d2_tile(q, rt):
    return (jnp.sum(q * q, axis=1, keepdims=True)
            - 2.0 * jnp.dot(q.astype(jnp.bfloat16), rt.astype(jnp.bfloat16),
                            preferred_element_type=F32)
            + jnp.sum(rt * rt, axis=0, keepdims=True))


def _knn_body(q_ref, rt_ref, idx_ref, d2_ref, *, k):
    d2 = _d2_tile(q_ref[...], rt_ref[...])
    iota = lax.broadcasted_iota(jnp.int32, d2.shape, 1)
    idx_cols, d_cols = [], []
    for _ in range(k):
        m = jnp.min(d2, axis=1, keepdims=True)
        j = jnp.argmin(d2, axis=1).reshape(-1, 1)
        idx_cols.append(j)
        d_cols.append(m)
        d2 = jnp.where(iota == j, jnp.inf, d2)
    idx_ref[...] = jnp.concatenate(idx_cols, axis=1)
    d2_ref[...] = jnp.concatenate(d_cols, axis=1)


def _knn_idx_body(q_ref, rt_ref, idx_ref, *, k):
    d2 = _d2_tile(q_ref[...], rt_ref[...])
    iota = lax.broadcasted_iota(jnp.int32, d2.shape, 1)
    idx_cols = []
    for _ in range(k):
        j = jnp.argmin(d2, axis=1).reshape(-1, 1)
        idx_cols.append(j)
        d2 = jnp.where(iota == j, jnp.inf, d2)
    idx_ref[...] = jnp.concatenate(idx_cols, axis=1)


def _knn_pad(q, nsrc):
    m = q.shape[0]
    cap = max(8, (int(1.5e6) // max(nsrc, 1)) // 8 * 8)
    grid = -(-m // cap)
    bq = _rup(-(-m // grid), 8)
    mp = bq * grid
    if mp > m:
        q = jnp.concatenate(
            [q, jnp.full((mp - m, q.shape[1]), 1e9, F32)], axis=0)
    return q, grid, bq, mp


def _knn(q, r, k):
    m, nsrc = q.shape[0], r.shape[0]
    q, grid, bq, mp = _knn_pad(q, nsrc)
    idx, d2 = pl.pallas_call(
        functools.partial(_knn_body, k=k),
        grid=(grid,),
        in_specs=[pl.BlockSpec((bq, 3), lambda i: (i, 0)),
                  pl.BlockSpec((3, nsrc), lambda i: (0, 0))],
        out_specs=[pl.BlockSpec((bq, k), lambda i: (i, 0)),
                   pl.BlockSpec((bq, k), lambda i: (i, 0))],
        out_shape=[jax.ShapeDtypeStruct((mp, k), jnp.int32),
                   jax.ShapeDtypeStruct((mp, k), F32)],
    )(q, r.T)
    return idx[:m], d2[:m]


def _knn_idx(q, r, k):
    m, nsrc = q.shape[0], r.shape[0]
    q, grid, bq, mp = _knn_pad(q, nsrc)
    idx = pl.pallas_call(
        functools.partial(_knn_idx_body, k=k),
        grid=(grid,),
        in_specs=[pl.BlockSpec((bq, 3), lambda i: (i, 0)),
                  pl.BlockSpec((3, nsrc), lambda i: (0, 0))],
        out_specs=pl.BlockSpec((bq, k), lambda i: (i, 0)),
        out_shape=jax.ShapeDtypeStruct((mp, k), jnp.int32),
    )(q, r.T)
    return idx[:m]


@functools.lru_cache(maxsize=None)
def _make_sc_gather(v, d, b):
    info = plsc.get_sparse_core_info()
    nw = info.num_cores * info.num_subcores
    bw = b // nw
    mesh = plsc.VectorSubcoreMesh(core_axis_name="c", subcore_axis_name="s")

    @functools.partial(
        pl.kernel, mesh=mesh,
        out_type=jax.ShapeDtypeStruct((b, d), F32),
        scratch_types=[pltpu.VMEM((bw,), jnp.int32),
                       pltpu.VMEM((128, d), F32),
                       pltpu.SemaphoreType.DMA],
    )
    def gather(table_hbm, idx_hbm, out_hbm, idx_v, rows_v, sem):
        wid = lax.axis_index("s") * info.num_cores + lax.axis_index("c")
        base = wid * bw
        pltpu.sync_copy(idx_hbm.at[pl.ds(base, bw)], idx_v)
        for j0 in range(0, bw, 128):
            sz = min(128, bw - j0)
            pltpu.async_copy(table_hbm.at[idx_v.at[pl.ds(j0, sz)]],
                             rows_v.at[pl.ds(0, sz)], sem).wait()
            pltpu.sync_copy(rows_v.at[pl.ds(0, sz)],
                            out_hbm.at[pl.ds(base + j0, sz)])

    return gather


def _gather_rows(table, flat_idx):
    n = flat_idx.shape[0]
    b = _rup(n, 256)
    if b > n:
        flat_idx = jnp.concatenate(
            [flat_idx, jnp.zeros((b - n,), jnp.int32)], axis=0)
    d = table.shape[1]
    dp = _rup(d, 128)
    if dp > d:
        table = jnp.pad(table, ((0, 0), (0, dp - d)))
    out = _make_sc_gather(table.shape[0], dp, b)(table, flat_idx)
    return out[:n, :d]


def _td_stats_body(g_ref, q_ref, wp_ref, hmax_ref, s1_ref, s2_ref, *, bm, k, cout):
    i = pl.program_id(0)
    g = g_ref[...]
    dp = g[:, :, :3] - q_ref[...][:, None, :]
    hp = jnp.dot(dp.reshape(bm * k, 3).astype(jnp.bfloat16),
                 wp_ref[...].astype(jnp.bfloat16),
                 preferred_element_type=F32)
    h = hp.reshape(bm, k, cout) + g[:, :, 3:]
    hmax_ref[...] = jnp.max(h, axis=1)
    ps1 = jnp.sum(jnp.sum(h, axis=1), axis=0, keepdims=True)
    ps2 = jnp.sum(jnp.sum(h * h, axis=1), axis=0, keepdims=True)

    @pl.when(i == 0)
    def _init():
        s1_ref[...] = ps1
        s2_ref[...] = ps2

    @pl.when(i > 0)
    def _acc():
        s1_ref[...] += ps1
        s2_ref[...] += ps2


def _td_norm_body(hmax_ref, s1_ref, s2_ref, o_ref, *, n):
    mu = s1_ref[...] / n
    var = s2_ref[...] / n - mu * mu
    o_ref[...] = jnp.maximum((hmax_ref[...] - mu) * lax.rsqrt(var + 1e-5), 0.0)


def _td_fuse(g, q, wp):
    m, k, dg = g.shape
    cout = dg - 3
    nb = max(1, -(-(m * k * dg * 4) // (3 * 2**20)))
    mp = _rup(m, 8 * nb)
    if mp > m:
        g = jnp.pad(g, ((0, mp - m), (0, 0), (0, 0)))
        q = jnp.pad(q, ((0, mp - m), (0, 0)))
    bm = mp // nb
    hmax, s1, s2 = pl.pallas_call(
        functools.partial(_td_stats_body, bm=bm, k=k, cout=cout),
        grid=(nb,),
        in_specs=[pl.BlockSpec((bm, k, dg), lambda i: (i, 0, 0)),
                  pl.BlockSpec((bm, 3), lambda i: (i, 0)),
                  pl.BlockSpec((3, cout), lambda i: (0, 0))],
        out_specs=[pl.BlockSpec((bm, cout), lambda i: (i, 0)),
                   pl.BlockSpec((1, cout), lambda i: (0, 0)),
                   pl.BlockSpec((1, cout), lambda i: (0, 0))],
        out_shape=[jax.ShapeDtypeStruct((mp, cout), F32),
                   jax.ShapeDtypeStruct((1, cout), F32),
                   jax.ShapeDtypeStruct((1, cout), F32)],
    )(g, q, wp)
    return pl.pallas_call(
        functools.partial(_td_norm_body, n=m * k),
        out_shape=jax.ShapeDtypeStruct((mp, cout), F32),
    )(hmax, s1, s2)[:m]


def _tu_fuse_body(x_ref, w_ref, b_ref, g0_ref, g1_ref, g2_ref, d2_ref, o_ref):
    h = jnp.dot(x_ref[...], w_ref[...], preferred_element_type=F32) + b_ref[...]
    mu = jnp.mean(h, axis=0, keepdims=True)
    var = jnp.mean(h * h, axis=0, keepdims=True) - mu * mu
    a = jnp.maximum((h - mu) * lax.rsqrt(var + 1e-5), 0.0)
    d2 = jnp.maximum(d2_ref[...], 0.0)
    w = 1.0 / (jnp.sqrt(d2) + 1e-8)
    w = w / jnp.sum(w, axis=1, keepdims=True)
    interp = (g0_ref[...] * w[:, 0:1] + g1_ref[...] * w[:, 1:2]
              + g2_ref[...] * w[:, 2:3])
    o_ref[...] = a + interp


def _tu_fuse(x, w, b, g, d2):
    n, cout = x.shape[0], w.shape[1]
    return pl.pallas_call(
        _tu_fuse_body,
        out_shape=jax.ShapeDtypeStruct((n, cout), F32),
    )(x, w, b.reshape(1, cout), g[:, 0, :], g[:, 1, :], g[:, 2, :], d2)


def _u5_body(x_ref, a_ref, bmat_ref, b1_ref, l2w_ref, l2b_ref, o_ref):
    x = x_ref[...]
    gmean = jnp.mean(x, axis=0, keepdims=True)
    g = jnp.maximum(
        jnp.dot(gmean, l2w_ref[...], preferred_element_type=F32) + l2b_ref[...],
        0.0)
    h = (jnp.dot(x, a_ref[...], preferred_element_type=F32)
         + jnp.dot(g, bmat_ref[...], preferred_element_type=F32)
         + b1_ref[...])
    mu = jnp.mean(h, axis=0, keepdims=True)
    var = jnp.mean(h * h, axis=0, keepdims=True) - mu * mu
    o_ref[...] = jnp.maximum((h - mu) * lax.rsqrt(var + 1e-5), 0.0)


def _u5(x5, l1w, l1b, l2w, l2b):
    cnt, c = x5.shape
    cout = l1w.shape[1]
    return pl.pallas_call(
        _u5_body,
        out_shape=jax.ShapeDtypeStruct((cnt, cout), F32),
    )(x5, l1w[:c], l1w[c:], l1b.reshape(1, cout), l2w, l2b.reshape(1, -1))


def _head_body(x_ref, w1_ref, b1_ref, w2_ref, b2_ref, o_ref):
    h = jnp.dot(x_ref[...], w1_ref[...], preferred_element_type=F32) + b1_ref[...]
    mu = jnp.mean(h, axis=0, keepdims=True)
    var = jnp.mean(h * h, axis=0, keepdims=True) - mu * mu
    h = jnp.maximum((h - mu) * lax.rsqrt(var + 1e-5), 0.0)
    o_ref[...] = jnp.dot(h, w2_ref[...], preferred_element_type=F32) + b2_ref[...]


def _head(x, w1, b1, w2, b2):
    n, cout = x.shape[0], w2.shape[1]
    return pl.pallas_call(
        _head_body,
        out_shape=jax.ShapeDtypeStruct((n, cout), F32),
    )(x, w1, b1.reshape(1, -1), w2, b2.reshape(1, cout))


def _transition_down(p, x, w, stride, k):
    nsrc = p.shape[0]
    m = nsrc // stride
    cout = w.shape[1]
    q = p[::stride][:m]
    zx = _zx(x, w[3:])
    table = jnp.concatenate([p, zx], axis=1)
    idx = _knn_idx(q, p, k)
    g = _gather_rows(table, idx.reshape(-1)).reshape(m, k, 3 + cout)
    return q, _td_fuse(g, q, w[:3])


def _transition_up(p1, x1, p2, x2, l1w, l1b, l2w, l2b):
    b2 = _lin_bn_relu(x2, l2w, l2b)
    idx, d2 = _knn(p1, p2, 3)
    g = _gather_rows(b2, idx.reshape(-1)).reshape(p1.shape[0], 3, -1)
    return _tu_fuse(x1, l1w, l1b, g, d2)


def kernel(coord, feat, offset, W1, W2, W3, W4, W5,
           U5_l1_W, U5_l1_b, U5_l2_W, U5_l2_b,
           U4_l1_W, U4_l1_b, U4_l2_W, U4_l2_b,
           U3_l1_W, U3_l1_b, U3_l2_W, U3_l2_b,
           U2_l1_W, U2_l1_b, U2_l2_W, U2_l2_b,
           U1_l1_W, U1_l1_b, U1_l2_W, U1_l2_b,
           C1_W, C1_b, C2_W, C2_b):
    del offset
    p1 = coord
    x1 = _lin_bn_relu(feat, W1, jnp.zeros((W1.shape[1],), F32))
    p2, x2 = _transition_down(p1, x1, W2, 4, 16)
    p3, x3 = _transition_down(p2, x2, W3, 4, 16)
    p4, x4 = _transition_down(p3, x3, W4, 4, 16)
    p5, x5 = _transition_down(p4, x4, W5, 4, 16)
    x5 = _u5(x5, U5_l1_W, U5_l1_b, U5_l2_W, U5_l2_b)
    x4 = _transition_up(p4, x4, p5, x5, U4_l1_W, U4_l1_b, U4_l2_W, U4_l2_b)
    x3 = _transition_up(p3, x3, p4, x4, U3_l1_W, U3_l1_b, U3_l2_W, U3_l2_b)
    x2 = _transition_up(p2, x2, p3, x3, U2_l1_W, U2_l1_b, U2_l2_W, U2_l2_b)
    x1 = _transition_up(p1, x1, p2, x2, U1_l1_W, U1_l1_b, U1_l2_W, U1_l2_b)
    return _head(x1, C1_W, C1_b, C2_W, C2_b)


# speedup vs baseline: 4.4199x; 1.0080x over previous
import functools

import jax
import jax.numpy as jnp
from jax import lax
from jax.experimental import pallas as pl
from jax.experimental.pallas import tpu as pltpu
from jax.experimental.pallas import tpu_sc as plsc

F32 = jnp.float32


def _rup(x, m):
    return (x + m - 1) // m * m


def _lin_bn_relu_body(x_ref, w_ref, b_ref, o_ref):
    h = jnp.dot(x_ref[...], w_ref[...], preferred_element_type=F32) + b_ref[...]
    mu = jnp.mean(h, axis=0, keepdims=True)
    var = jnp.mean(h * h, axis=0, keepdims=True) - mu * mu
    o_ref[...] = jnp.maximum((h - mu) * lax.rsqrt(var + 1e-5), 0.0)


def _lin_bn_relu(x, w, b):
    n, cout = x.shape[0], w.shape[1]
    return pl.pallas_call(
        _lin_bn_relu_body,
        out_shape=jax.ShapeDtypeStruct((n, cout), F32),
    )(x, w, b.reshape(1, cout))


def _zx_body(x_ref, wx_ref, z_ref):
    z_ref[...] = jnp.dot(x_ref[...], wx_ref[...], preferred_element_type=F32)


def _zx(x, wx):
    n, cout = x.shape[0], wx.shape[1]
    return pl.pallas_call(
        _zx_body,
        out_shape=jax.ShapeDtypeStruct((n, cout), F32),
    )(x, wx)


def _d2_tile(q, rt):
    return (jnp.sum(q * q, axis=1, keepdims=True)
            - 2.0 * jnp.dot(q.astype(jnp.bfloat16), rt.astype(jnp.bfloat16),
                            preferred_element_type=F32)
            + jnp.sum(rt * rt, axis=0, keepdims=True))


def _knn_body(q_ref, rt_ref, idx_ref, d2_ref, *, k):
    d2 = _d2_tile(q_ref[...], rt_ref[...])
    iota = lax.broadcasted_iota(jnp.int32, d2.shape, 1)
    idx_cols, d_cols = [], []
    for _ in range(k):
        m = jnp.min(d2, axis=1, keepdims=True)
        j = jnp.argmin(d2, axis=1).reshape(-1, 1)
        idx_cols.append(j)
        d_cols.append(m)
        d2 = jnp.where(iota == j, jnp.inf, d2)
    idx_ref[...] = jnp.concatenate(idx_cols, axis=1)
    d2_ref[...] = jnp.concatenate(d_cols, axis=1)


def _knn_idx_body(q_ref, rt_ref, idx_ref, *, k):
    d2 = _d2_tile(q_ref[...], rt_ref[...])
    iota = lax.broadcasted_iota(jnp.int32, d2.shape, 1)
    idx_cols = []
    for _ in range(k):
        j = jnp.argmin(d2, axis=1).reshape(-1, 1)
        idx_cols.append(j)
        d2 = jnp.where(iota == j, jnp.inf, d2)
    idx_ref[...] = jnp.concatenate(idx_cols, axis=1)


def _knn_pad(q, nsrc):
    m = q.shape[0]
    cap = max(8, (int(1.5e6) // max(nsrc, 1)) // 8 * 8)
    grid = -(-m // cap)
    bq = _rup(-(-m // grid), 8)
    mp = bq * grid
    if mp > m:
        q = jnp.concatenate(
            [q, jnp.full((mp - m, q.shape[1]), 1e9, F32)], axis=0)
    return q, grid, bq, mp


def _knn(q, r, k):
    m, nsrc = q.shape[0], r.shape[0]
    q, grid, bq, mp = _knn_pad(q, nsrc)
    idx, d2 = pl.pallas_call(
        functools.partial(_knn_body, k=k),
        grid=(grid,),
        in_specs=[pl.BlockSpec((bq, 3), lambda i: (i, 0)),
                  pl.BlockSpec((3, nsrc), lambda i: (0, 0))],
        out_specs=[pl.BlockSpec((bq, k), lambda i: (i, 0)),
                   pl.BlockSpec((bq, k), lambda i: (i, 0))],
        out_shape=[jax.ShapeDtypeStruct((mp, k), jnp.int32),
                   jax.ShapeDtypeStruct((mp, k), F32)],
    )(q, r.T)
    return idx[:m], d2[:m]


def _knn_idx(q, r, k):
    m, nsrc = q.shape[0], r.shape[0]
    q, grid, bq, mp = _knn_pad(q, nsrc)
    idx = pl.pallas_call(
        functools.partial(_knn_idx_body, k=k),
        grid=(grid,),
        in_specs=[pl.BlockSpec((bq, 3), lambda i: (i, 0)),
                  pl.BlockSpec((3, nsrc), lambda i: (0, 0))],
        out_specs=pl.BlockSpec((bq, k), lambda i: (i, 0)),
        out_shape=jax.ShapeDtypeStruct((mp, k), jnp.int32),
    )(q, r.T)
    return idx[:m]


@functools.lru_cache(maxsize=None)
def _make_sc_gather(v, d, b):
    info = plsc.get_sparse_core_info()
    nw = info.num_cores * info.num_subcores
    bw = b // nw
    mesh = plsc.VectorSubcoreMesh(core_axis_name="c", subcore_axis_name="s")

    @functools.partial(
        pl.kernel, mesh=mesh,
        out_type=jax.ShapeDtypeStruct((b, d), F32),
        scratch_types=[pltpu.VMEM((bw,), jnp.int32),
                       pltpu.VMEM((128, d), F32),
                       pltpu.VMEM((128, d), F32),
                       pltpu.SemaphoreType.DMA,
                       pltpu.SemaphoreType.DMA,
                       pltpu.SemaphoreType.DMA,
                       pltpu.SemaphoreType.DMA],
    )
    def gather(table_hbm, idx_hbm, out_hbm, idx_v, rows0, rows1, g0, g1, w0, w1):
        wid = lax.axis_index("s") * info.num_cores + lax.axis_index("c")
        base = wid * bw
        pltpu.sync_copy(idx_hbm.at[pl.ds(base, bw)], idx_v)
        bufs, gsem, wsem = (rows0, rows1), (g0, g1), (w0, w1)
        wops = [None, None]
        prev = None
        for c in range(0, -(-bw // 128)):
            j0 = c * 128
            sz = min(128, bw - j0)
            p = c & 1
            if wops[p] is not None:
                wops[p].wait()
            g = pltpu.async_copy(table_hbm.at[idx_v.at[pl.ds(j0, sz)]],
                                 bufs[p].at[pl.ds(0, sz)], gsem[p])
            if prev is not None:
                gp, pp, szp, j0p = prev
                gp.wait()
                wops[pp] = pltpu.async_copy(
                    bufs[pp].at[pl.ds(0, szp)],
                    out_hbm.at[pl.ds(base + j0p, szp)], wsem[pp])
            prev = (g, p, sz, j0)
        g, p, sz, j0 = prev
        g.wait()
        pltpu.sync_copy(bufs[p].at[pl.ds(0, sz)],
                        out_hbm.at[pl.ds(base + j0, sz)])

    return gather


def _gather_rows(table, flat_idx):
    n = flat_idx.shape[0]
    b = _rup(n, 256)
    if b > n:
        flat_idx = jnp.concatenate(
            [flat_idx, jnp.zeros((b - n,), jnp.int32)], axis=0)
    d = table.shape[1]
    dp = _rup(d, 128)
    if dp > d:
        table = jnp.pad(table, ((0, 0), (0, dp - d)))
    out = _make_sc_gather(table.shape[0], dp, b)(table, flat_idx)
    return out[:n, :d]


def _td_stats_body(g_ref, q_ref, wp_ref, hmax_ref, s1_ref, s2_ref, *, bm, k, cout):
    i = pl.program_id(0)
    g = g_ref[...]
    dp = g[:, :, :3] - q_ref[...][:, None, :]
    hp = jnp.dot(dp.reshape(bm * k, 3).astype(jnp.bfloat16),
                 wp_ref[...].astype(jnp.bfloat16),
                 preferred_element_type=F32)
    h = hp.reshape(bm, k, cout) + g[:, :, 3:]
    hmax_ref[...] = jnp.max(h, axis=1)
    ps1 = jnp.sum(jnp.sum(h, axis=1), axis=0, keepdims=True)
    ps2 = jnp.sum(jnp.sum(h * h, axis=1), axis=0, keepdims=True)

    @pl.when(i == 0)
    def _init():
        s1_ref[...] = ps1
        s2_ref[...] = ps2

    @pl.when(i > 0)
    def _acc():
        s1_ref[...] += ps1
        s2_ref[...] += ps2


def _td_norm_body(hmax_ref, s1_ref, s2_ref, o_ref, *, n):
    mu = s1_ref[...] / n
    var = s2_ref[...] / n - mu * mu
    o_ref[...] = jnp.maximum((hmax_ref[...] - mu) * lax.rsqrt(var + 1e-5), 0.0)


def _td_fuse(g, q, wp):
    m, k, dg = g.shape
    cout = dg - 3
    nb = max(1, -(-(m * k * dg * 4) // (3 * 2**20)))
    mp = _rup(m, 8 * nb)
    if mp > m:
        g = jnp.pad(g, ((0, mp - m), (0, 0), (0, 0)))
        q = jnp.pad(q, ((0, mp - m), (0, 0)))
    bm = mp // nb
    hmax, s1, s2 = pl.pallas_call(
        functools.partial(_td_stats_body, bm=bm, k=k, cout=cout),
        grid=(nb,),
        in_specs=[pl.BlockSpec((bm, k, dg), lambda i: (i, 0, 0)),
                  pl.BlockSpec((bm, 3), lambda i: (i, 0)),
                  pl.BlockSpec((3, cout), lambda i: (0, 0))],
        out_specs=[pl.BlockSpec((bm, cout), lambda i: (i, 0)),
                   pl.BlockSpec((1, cout), lambda i: (0, 0)),
                   pl.BlockSpec((1, cout), lambda i: (0, 0))],
        out_shape=[jax.ShapeDtypeStruct((mp, cout), F32),
                   jax.ShapeDtypeStruct((1, cout), F32),
                   jax.ShapeDtypeStruct((1, cout), F32)],
    )(g, q, wp)
    return pl.pallas_call(
        functools.partial(_td_norm_body, n=m * k),
        out_shape=jax.ShapeDtypeStruct((mp, cout), F32),
    )(hmax, s1, s2)[:m]


def _tu_fuse_body(x_ref, w_ref, b_ref, g0_ref, g1_ref, g2_ref, d2_ref, o_ref):
    h = jnp.dot(x_ref[...], w_ref[...], preferred_element_type=F32) + b_ref[...]
    mu = jnp.mean(h, axis=0, keepdims=True)
    var = jnp.mean(h * h, axis=0, keepdims=True) - mu * mu
    a = jnp.maximum((h - mu) * lax.rsqrt(var + 1e-5), 0.0)
    d2 = jnp.maximum(d2_ref[...], 0.0)
    w = 1.0 / (jnp.sqrt(d2) + 1e-8)
    w = w / jnp.sum(w, axis=1, keepdims=True)
    interp = (g0_ref[...] * w[:, 0:1] + g1_ref[...] * w[:, 1:2]
              + g2_ref[...] * w[:, 2:3])
    o_ref[...] = a + interp


def _tu_fuse(x, w, b, g, d2):
    n, cout = x.shape[0], w.shape[1]
    return pl.pallas_call(
        _tu_fuse_body,
        out_shape=jax.ShapeDtypeStruct((n, cout), F32),
    )(x, w, b.reshape(1, cout), g[:, 0, :], g[:, 1, :], g[:, 2, :], d2)


def _u5_body(x_ref, a_ref, bmat_ref, b1_ref, l2w_ref, l2b_ref, o_ref):
    x = x_ref[...]
    gmean = jnp.mean(x, axis=0, keepdims=True)
    g = jnp.maximum(
        jnp.dot(gmean, l2w_ref[...], preferred_element_type=F32) + l2b_ref[...],
        0.0)
    h = (jnp.dot(x, a_ref[...], preferred_element_type=F32)
         + jnp.dot(g, bmat_ref[...], preferred_element_type=F32)
         + b1_ref[...])
    mu = jnp.mean(h, axis=0, keepdims=True)
    var = jnp.mean(h * h, axis=0, keepdims=True) - mu * mu
    o_ref[...] = jnp.maximum((h - mu) * lax.rsqrt(var + 1e-5), 0.0)


def _u5(x5, l1w, l1b, l2w, l2b):
    cnt, c = x5.shape
    cout = l1w.shape[1]
    return pl.pallas_call(
        _u5_body,
        out_shape=jax.ShapeDtypeStruct((cnt, cout), F32),
    )(x5, l1w[:c], l1w[c:], l1b.reshape(1, cout), l2w, l2b.reshape(1, -1))


def _head_body(x_ref, w1_ref, b1_ref, w2_ref, b2_ref, o_ref):
    h = jnp.dot(x_ref[...], w1_ref[...], preferred_element_type=F32) + b1_ref[...]
    mu = jnp.mean(h, axis=0, keepdims=True)
    var = jnp.mean(h * h, axis=0, keepdims=True) - mu * mu
    h = jnp.maximum((h - mu) * lax.rsqrt(var + 1e-5), 0.0)
    o_ref[...] = jnp.dot(h, w2_ref[...], preferred_element_type=F32) + b2_ref[...]


def _head(x, w1, b1, w2, b2):
    n, cout = x.shape[0], w2.shape[1]
    return pl.pallas_call(
        _head_body,
        out_shape=jax.ShapeDtypeStruct((n, cout), F32),
    )(x, w1, b1.reshape(1, -1), w2, b2.reshape(1, cout))


def _transition_down(p, x, w, stride, k):
    nsrc = p.shape[0]
    m = nsrc // stride
    cout = w.shape[1]
    q = p[::stride][:m]
    zx = _zx(x, w[3:])
    table = jnp.concatenate([p, zx], axis=1)
    idx = _knn_idx(q, p, k)
    g = _gather_rows(table, idx.reshape(-1)).reshape(m, k, 3 + cout)
    return q, _td_fuse(g, q, w[:3])


def _transition_up(p1, x1, p2, x2, l1w, l1b, l2w, l2b):
    b2 = _lin_bn_relu(x2, l2w, l2b)
    idx, d2 = _knn(p1, p2, 3)
    g = _gather_rows(b2, idx.reshape(-1)).reshape(p1.shape[0], 3, -1)
    return _tu_fuse(x1, l1w, l1b, g, d2)


def kernel(coord, feat, offset, W1, W2, W3, W4, W5,
           U5_l1_W, U5_l1_b, U5_l2_W, U5_l2_b,
           U4_l1_W, U4_l1_b, U4_l2_W, U4_l2_b,
           U3_l1_W, U3_l1_b, U3_l2_W, U3_l2_b,
           U2_l1_W, U2_l1_b, U2_l2_W, U2_l2_b,
           U1_l1_W, U1_l1_b, U1_l2_W, U1_l2_b,
           C1_W, C1_b, C2_W, C2_b):
    del offset
    p1 = coord
    x1 = _lin_bn_relu(feat, W1, jnp.zeros((W1.shape[1],), F32))
    p2, x2 = _transition_down(p1, x1, W2, 4, 16)
    p3, x3 = _transition_down(p2, x2, W3, 4, 16)
    p4, x4 = _transition_down(p3, x3, W4, 4, 16)
    p5, x5 = _transition_down(p4, x4, W5, 4, 16)
    x5 = _u5(x5, U5_l1_W, U5_l1_b, U5_l2_W, U5_l2_b)
    x4 = _transition_up(p4, x4, p5, x5, U4_l1_W, U4_l1_b, U4_l2_W, U4_l2_b)
    x3 = _transition_up(p3, x3, p4, x4, U3_l1_W, U3_l1_b, U3_l2_W, U3_l2_b)
    x2 = _transition_up(p2, x2, p3, x3, U2_l1_W, U2_l1_b, U2_l2_W, U2_l2_b)
    x1 = _transition_up(p1, x1, p2, x2, U1_l1_W, U1_l1_b, U1_l2_W, U1_l2_b)
    return _head(x1, C1_W, C1_b, C2_W, C2_b)
